# baseline (speedup 1.0000x reference)
import functools

import jax
import jax.numpy as jnp
from jax import lax
from jax.experimental import pallas as pl
from jax.experimental.pallas import tpu as pltpu
from jax.experimental.pallas import tpu_sc as plsc

D_MODEL = 64
CH = 128
NBUF = 8
LEAD = 4

NUM_CORES = 2
NUM_SUBCORES = 16
NUM_WORKERS = NUM_CORES * NUM_SUBCORES


@functools.lru_cache(maxsize=None)
def _build(B, D):
    b_per_w = B // NUM_WORKERS
    n_ch = b_per_w // CH
    n_groups = n_ch // NBUF
    assert b_per_w * NUM_WORKERS == B and n_ch * CH == b_per_w
    assert n_groups * NBUF == n_ch and n_groups >= 3

    mesh = plsc.VectorSubcoreMesh(core_axis_name="c", subcore_axis_name="s")

    @functools.partial(
        pl.kernel,
        mesh=mesh,
        out_type=jax.ShapeDtypeStruct((B, D), jnp.float32),
        compiler_params=pltpu.CompilerParams(use_tc_tiling_on_sc=False),
        scratch_types=[
            pltpu.VMEM((n_ch, CH), jnp.int32),
            pltpu.VMEM((NBUF, CH, D), jnp.float32),
            pltpu.SemaphoreType.DMA((NBUF,)),
            pltpu.SemaphoreType.DMA((NBUF,)),
        ],
    )
    def emb(x_hbm, lut_hbm, out_hbm, idx_v, rows_v, gsem, osem):
        wid = lax.axis_index("s") * NUM_CORES + lax.axis_index("c")
        base = wid * b_per_w

        pltpu.sync_copy(x_hbm.at[wid], idx_v)

        def start_gather(j, b):
            pltpu.make_async_copy(
                lut_hbm.at[idx_v.at[j]], rows_v.at[b], gsem.at[b]
            ).start()

        def wait_gather(b):
            pltpu.make_async_copy(
                lut_hbm.at[idx_v.at[0]], rows_v.at[b], gsem.at[b]
            ).wait()

        def start_write(j, b):
            pltpu.make_async_copy(
                rows_v.at[b], out_hbm.at[pl.ds(base + j * CH, CH)], osem.at[b]
            ).start()

        def wait_write(b):
            pltpu.make_async_copy(
                rows_v.at[b], out_hbm.at[pl.ds(base, CH)], osem.at[b]
            ).wait()

        def step(j, b, wait_prev_write, gather_ahead):
            bg = (b + LEAD) % NBUF
            if wait_prev_write:
                wait_write(bg)
            if gather_ahead:
                start_gather(j + LEAD, bg)
            wait_gather(b)
            start_write(j, b)

        for b in range(LEAD):
            start_gather(b, b)

        for b in range(NBUF):
            step(b, b, wait_prev_write=(b >= NBUF - LEAD), gather_ahead=True)

        def group(g, carry):
            for b in range(NBUF):
                step(g * NBUF + b, b, wait_prev_write=True, gather_ahead=True)
            return carry

        lax.fori_loop(1, n_groups - 1, group, 0)

        for b in range(NBUF):
            j = n_ch - NBUF + b
            step(j, b, wait_prev_write=True, gather_ahead=(b < NBUF - LEAD))

        for j in range(n_ch - (NBUF - LEAD), n_ch):
            wait_write(j % NBUF)

    return emb


def kernel(x, lut):
    batch, seq = x.shape
    B = batch * seq
    b_per_w = B // NUM_WORKERS
    x_tiles = x.reshape(NUM_WORKERS, b_per_w // CH, CH).astype(jnp.int32)
    out = _build(B, lut.shape[1])(x_tiles, lut)
    return out.reshape(batch, seq, lut.shape[1])
